# Initial kernel scaffold; baseline (speedup 1.0000x reference)
import jax
import jax.numpy as jnp
from jax.experimental import pallas as pl


def kernel(predictions, inputs, heart):
    raise NotImplementedError("write your pallas kernel here")


# speedup vs baseline: 5.3912x; 5.3912x over previous
import functools

import jax
import jax.numpy as jnp
from jax.experimental import pallas as pl
from jax.experimental.pallas import tpu as pltpu

_EPS = 1e-10
_K = 4
_M = 3


def _gmm_body(pred_ref, inp_ref, heart_ref, out_ref):
    p = pred_ref[0]
    x = inp_ref[0]
    h = heart_ref[0]
    h2 = h[0]

    pm = p * h

    denom = jnp.sum(pm, axis=(1, 2)) + _EPS
    s1 = []
    s2 = []
    for m in range(_M):
        xm = x[m]
        pxm = pm * xm
        s1.append(jnp.sum(pxm, axis=(1, 2)))
        s2.append(jnp.sum(pxm * xm, axis=(1, 2)))
    s1 = jnp.stack(s1, axis=1)
    s2 = jnp.stack(s2, axis=1)

    mu = s1 / denom[:, None]
    var = s2 / denom[:, None] - mu * mu + _EPS
    inv2 = 0.5 / var
    logcoef = -0.5 * jnp.sum(jnp.log(2.0 * jnp.pi * var), axis=1)

    acc = jnp.zeros_like(h2)
    for k in range(_K):
        q = logcoef[k] - (
            (x[0] - mu[k, 0]) ** 2 * inv2[k, 0]
            + (x[1] - mu[k, 1]) ** 2 * inv2[k, 1]
            + (x[2] - mu[k, 2]) ** 2 * inv2[k, 2]
        )
        acc = acc + p[k] * jnp.exp(q)
    ll = jnp.log(acc + _EPS)

    num = jnp.sum(h2 * ll)
    den = jnp.sum(h2)
    out_ref[pl.program_id(0)] = -num / den


@functools.partial(jax.jit, static_argnames=())
def kernel(predictions, inputs, heart):
    B, K, X, Y = predictions.shape
    losses = pl.pallas_call(
        _gmm_body,
        grid=(B,),
        in_specs=[
            pl.BlockSpec((1, K, X, Y), lambda b: (b, 0, 0, 0)),
            pl.BlockSpec((1, inputs.shape[1], X, Y), lambda b: (b, 0, 0, 0)),
            pl.BlockSpec((1, 1, X, Y), lambda b: (b, 0, 0, 0)),
        ],
        out_specs=pl.BlockSpec(memory_space=pltpu.MemorySpace.SMEM),
        out_shape=jax.ShapeDtypeStruct((B,), jnp.float32),
    )(predictions, inputs, heart)
    return jnp.mean(losses)
